# Initial kernel scaffold; baseline (speedup 1.0000x reference)
import jax
import jax.numpy as jnp
from jax.experimental import pallas as pl


def kernel(adj_indices, adj_values, ent_table, rel_table, W1, W2):
    raise NotImplementedError("write your pallas kernel here")


# speedup vs baseline: 4.2564x; 4.2564x over previous
import functools

import jax
import jax.numpy as jnp
from jax import lax
from jax.experimental import pallas as pl
from jax.experimental.pallas import tpu as pltpu
import jax.experimental.pallas.tpu_sc as plsc

N = 10000
NPAD = 10240
E = 320000
D = 128
R = 500

NUM_CORES = 2
NUM_SUBCORES = 16
NW = NUM_CORES * NUM_SUBCORES
EDGES_PER_W = E // NW
CHUNK = 80
NCHUNKS = EDGES_PER_W // CHUNK
ROWS_PER_SUB = NPAD // NUM_SUBCORES
ZROWS = 128


def _spmm_body(cols_hbm, dsts_hbm, vals_hbm, x_hbm, out_hbm,
               col_v, dst_v, val_v, rows_v, zbuf, acc, sem):
    cid = lax.axis_index("c")
    sid = lax.axis_index("s")
    w = cid * NUM_SUBCORES + sid

    def zero_body(i, _):
        for j in range(8):
            zbuf[i, pl.ds(j * 16, 16)] = jnp.zeros((16,), jnp.float32)
        return 0
    lax.fori_loop(0, ZROWS, zero_body, 0)
    for k in range(ROWS_PER_SUB // ZROWS):
        pltpu.sync_copy(zbuf, acc.at[pl.ds(sid * ROWS_PER_SUB + k * ZROWS, ZROWS)])
    plsc.subcore_barrier()

    def chunk_body(ci, _):
        base = w * EDGES_PER_W + ci * CHUNK
        pltpu.sync_copy(cols_hbm.at[pl.ds(base, CHUNK)], col_v)
        pltpu.sync_copy(vals_hbm.at[pl.ds(base, CHUNK)], val_v)
        pltpu.sync_copy(dsts_hbm.at[pl.ds(base, CHUNK)], dst_v)
        pltpu.async_copy(x_hbm.at[col_v], rows_v, sem).wait()

        def g_body(g, _):
            v16 = val_v[pl.ds(g * 16, 16)]
            for i in range(16):
                e = g * 16 + i
                bv = jnp.full((16,), v16[i], dtype=jnp.float32)
                for j in range(8):
                    sl = pl.ds(j * 16, 16)
                    rows_v[e, sl] = rows_v[e, sl] * bv
            return 0
        lax.fori_loop(0, CHUNK // 16, g_body, 0)

        pltpu.sync_copy(rows_v, acc.at[dst_v], add=True)
        return 0
    lax.fori_loop(0, NCHUNKS, chunk_body, 0)

    plsc.subcore_barrier()
    pltpu.sync_copy(acc.at[pl.ds(sid * ROWS_PER_SUB, ROWS_PER_SUB)],
                    out_hbm.at[cid, pl.ds(sid * ROWS_PER_SUB, ROWS_PER_SUB)])


_spmm = pl.kernel(
    _spmm_body,
    out_type=jax.ShapeDtypeStruct((NUM_CORES, NPAD, D), jnp.float32),
    mesh=plsc.VectorSubcoreMesh(core_axis_name="c", subcore_axis_name="s"),
    scratch_types=[
        pltpu.VMEM((CHUNK,), jnp.int32),
        pltpu.VMEM((CHUNK,), jnp.int32),
        pltpu.VMEM((CHUNK,), jnp.float32),
        pltpu.VMEM((CHUNK, D), jnp.float32),
        pltpu.VMEM((ZROWS, D), jnp.float32),
        pltpu.VMEM_SHARED((NPAD, D), jnp.float32),
        pltpu.SemaphoreType.DMA,
    ],
)


def _mm1_body(x_ref, w_ref, o_ref):
    o_ref[...] = jnp.dot(x_ref[...], w_ref[...],
                         preferred_element_type=jnp.float32)


def _mm2_body(p_ref, w_ref, o_ref):
    x = jnp.maximum(p_ref[0] + p_ref[1], 0.0)
    o_ref[...] = jnp.dot(x, w_ref[...], preferred_element_type=jnp.float32)


def _norm_body(p_ref, rel_ref, ent_o, rel_o):
    x = p_ref[0] + p_ref[1]
    n = jnp.sqrt(jnp.sum(x * x, axis=-1, keepdims=True))
    ent_o[...] = x / jnp.maximum(n, 1e-12)
    r = rel_ref[...]
    nr = jnp.sqrt(jnp.sum(r * r, axis=-1, keepdims=True))
    rel_o[...] = r / jnp.maximum(nr, 1e-12)


def kernel(adj_indices, adj_values, ent_table, rel_table, W1, W2):
    dsts = adj_indices[0]
    cols = adj_indices[1]

    support1 = pl.pallas_call(
        _mm1_body,
        out_shape=jax.ShapeDtypeStruct((N, D), jnp.float32),
    )(ent_table, W1)

    p1 = _spmm(cols, dsts, adj_values, support1)

    support2 = pl.pallas_call(
        _mm2_body,
        out_shape=jax.ShapeDtypeStruct((NPAD, D), jnp.float32),
    )(p1, W2)

    p2 = _spmm(cols, dsts, adj_values, support2)

    ent_emb, rel_emb = pl.pallas_call(
        _norm_body,
        out_shape=(jax.ShapeDtypeStruct((NPAD, D), jnp.float32),
                   jax.ShapeDtypeStruct((R, D), jnp.float32)),
    )(p2, rel_table)

    return (ent_emb[:N], rel_emb)
